# baseline (speedup 1.0000x reference)
import functools

import jax
import jax.numpy as jnp
from jax import lax
from jax.experimental import pallas as pl
from jax.experimental.pallas import tpu as pltpu
from jax.experimental.pallas import tpu_sc as plsc

NUM_ENTITIES = 1000000
NUM_RELATIONS = 1000
EMBED_DIM = 32
BATCH = 16384

NC = 2
NS = 16
LANES = 16
NW = NC * NS
BPW = BATCH // NW
CHUNK = 128
NCHUNK = BPW // CHUNK


def _body(heads_hbm, rels_hbm, tails_hbm, ent_hbm, rel_hbm, out_hbm,
          idx_h, idx_r, idx_t, rows_h, rows_r, rows_t, out_v, sem):
    wid = lax.axis_index("s") * NC + lax.axis_index("c")
    base = wid * BPW

    for j in range(NCHUNK):
        src = pl.ds(base + j * CHUNK, CHUNK)
        pltpu.sync_copy(heads_hbm.at[src], idx_h.at[j])
        pltpu.sync_copy(rels_hbm.at[src], idx_r.at[j])
        pltpu.sync_copy(tails_hbm.at[src], idx_t.at[j])

    copies = []
    for j in range(NCHUNK):
        dst = pl.ds(j * CHUNK, CHUNK)
        copies.append(pltpu.async_copy(ent_hbm.at[idx_h.at[j]], rows_h.at[dst], sem))
        copies.append(pltpu.async_copy(rel_hbm.at[idx_r.at[j]], rows_r.at[dst], sem))
        copies.append(pltpu.async_copy(ent_hbm.at[idx_t.at[j]], rows_t.at[dst], sem))
    for c in copies:
        c.wait()

    lane = lax.iota(jnp.int32, LANES)

    def group(g, carry):
        acc = jnp.zeros((LANES,), jnp.float32)
        for u in range(LANES):
            i = g * LANES + u
            x = jnp.zeros((LANES,), jnp.float32)
            for half in range(EMBED_DIM // LANES):
                sl = pl.ds(half * LANES, LANES)
                x = x + rows_h[i, sl] * rows_r[i, sl] * rows_t[i, sl]
            acc = jnp.where(lane == u, jnp.sum(x), acc)
        out_v[pl.ds(g * LANES, LANES)] = acc
        return carry

    lax.fori_loop(0, BPW // LANES, group, 0)

    pltpu.sync_copy(out_v, out_hbm.at[pl.ds(base, BPW)])


@functools.partial(jax.jit, static_argnames=())
def _run(heads, relations, tails, entity_table, relation_table):
    mesh = plsc.VectorSubcoreMesh(core_axis_name="c", subcore_axis_name="s")
    k = pl.kernel(
        _body,
        out_type=jax.ShapeDtypeStruct((BATCH,), jnp.float32),
        mesh=mesh,
        compiler_params=pltpu.CompilerParams(
            needs_layout_passes=False, use_tc_tiling_on_sc=False),
        scratch_types=[
            pltpu.VMEM((NCHUNK, CHUNK), jnp.int32),
            pltpu.VMEM((NCHUNK, CHUNK), jnp.int32),
            pltpu.VMEM((NCHUNK, CHUNK), jnp.int32),
            pltpu.VMEM((BPW, EMBED_DIM), jnp.float32),
            pltpu.VMEM((BPW, EMBED_DIM), jnp.float32),
            pltpu.VMEM((BPW, EMBED_DIM), jnp.float32),
            pltpu.VMEM((BPW,), jnp.float32),
            pltpu.SemaphoreType.DMA,
        ],
    )
    return k(heads, relations, tails, entity_table, relation_table)


def kernel(heads, relations, tails, entity_table, relation_table):
    return _run(
        heads.astype(jnp.int32),
        relations.astype(jnp.int32),
        tails.astype(jnp.int32),
        entity_table,
        relation_table,
    )
